# Initial kernel scaffold; baseline (speedup 1.0000x reference)
import jax
import jax.numpy as jnp
from jax.experimental import pallas as pl


def kernel(x, params):
    raise NotImplementedError("write your pallas kernel here")


# speedup vs baseline: 2.7591x; 2.7591x over previous
import functools
import math

import jax
import jax.numpy as jnp
from jax.experimental import pallas as pl
from jax.experimental.pallas import tpu as pltpu

D = 512
H = 8
DH = D // H
T = 512
B = 8
NEG = -1e9


def _ln(x, g, b):
    m = x.mean(-1, keepdims=True)
    v = ((x - m) ** 2).mean(-1, keepdims=True)
    return (x - m) / jnp.sqrt(v + 1e-5) * g + b


def _gelu_exact(x):
    return 0.5 * x * (1.0 + jax.lax.erf(x * (1.0 / math.sqrt(2.0))))


def _scale_kernel(x_ref, mask_ref, wq_ref, wk_ref, wv_ref, bq_ref, bk_ref,
                  bv_ref, wo_ref, bo_ref, g1_ref, b1_ref, w1_ref, bf1_ref,
                  w2_ref, bf2_ref, g2_ref, b2_ref, out_ref):
    xb = x_ref[0]
    mask_row = mask_ref[0]
    scale = 1.0 / math.sqrt(DH)
    acc = jnp.zeros((T, D), dtype=jnp.float32)
    for h in range(H):
        q = jnp.dot(xb, wq_ref[h], preferred_element_type=jnp.float32) + bq_ref[h]
        k = jnp.dot(xb, wk_ref[h], preferred_element_type=jnp.float32) + bk_ref[h]
        v = jnp.dot(xb, wv_ref[h], preferred_element_type=jnp.float32) + bv_ref[h]
        logits = jax.lax.dot_general(
            q, k, (((1,), (1,)), ((), ())),
            preferred_element_type=jnp.float32) * scale
        logits = jnp.where(mask_row > 0.5, NEG, logits)
        m = jnp.max(logits, axis=-1, keepdims=True)
        e = jnp.exp(logits - m)
        a = e / jnp.sum(e, axis=-1, keepdims=True)
        o = jnp.dot(a, v, preferred_element_type=jnp.float32)
        acc = acc + jnp.dot(o, wo_ref[h], preferred_element_type=jnp.float32)
    att = acc + bo_ref[0]
    x1 = _ln(xb + att, g1_ref[0], b1_ref[0])
    f = jnp.dot(x1, w1_ref[...], preferred_element_type=jnp.float32) + bf1_ref[0]
    f = _gelu_exact(f)
    f = jnp.dot(f, w2_ref[...], preferred_element_type=jnp.float32) + bf2_ref[0]
    out_ref[0] = _ln(x1 + f, g2_ref[0], b2_ref[0])


def _scale_block(x, p, mask):
    wqkv = p['Wqkv']
    wq, wk, wv = jnp.split(wqkv, 3, axis=1)
    def heads(w):
        return w.reshape(D, H, DH).transpose(1, 0, 2)
    wqh, wkh, wvh = heads(wq), heads(wk), heads(wv)
    bq, bk, bv = jnp.split(p['bqkv'], 3)
    bqh = bq.reshape(H, 1, DH)
    bkh = bk.reshape(H, 1, DH)
    bvh = bv.reshape(H, 1, DH)
    woh = p['Wo'].reshape(H, DH, D)
    row = lambda a: a.reshape(1, -1)
    const = lambda shape: pl.BlockSpec(shape, lambda b: (0,) * len(shape))
    out = pl.pallas_call(
        _scale_kernel,
        grid=(B,),
        in_specs=[
            pl.BlockSpec((1, T, D), lambda b: (b, 0, 0)),
            pl.BlockSpec((1, 1, T), lambda b: (b, 0, 0)),
            const((H, D, DH)), const((H, D, DH)), const((H, D, DH)),
            const((H, 1, DH)), const((H, 1, DH)), const((H, 1, DH)),
            const((H, DH, D)), const((1, D)),
            const((1, D)), const((1, D)),
            const((D, 4 * D)), const((1, 4 * D)),
            const((4 * D, D)), const((1, D)),
            const((1, D)), const((1, D)),
        ],
        out_specs=pl.BlockSpec((1, T, D), lambda b: (b, 0, 0)),
        out_shape=jax.ShapeDtypeStruct((B, T, D), jnp.float32),
    )(x, mask, wqh, wkh, wvh, bqh, bkh, bvh, woh, row(p['bo']),
      row(p['g1']), row(p['b1']), p['W1'], row(p['bf1']), p['W2'],
      row(p['bf2']), row(p['g2']), row(p['b2']))
    return out


def _bind_agg_kernel(x_ref, wkb_ref, bkb_ref, wqb_ref, bqb_ref, w1_ref, b1_ref,
                     g_ref, bn_ref, w2_ref, b2_ref, ch_ref, pad_ref, s_ref):
    xb = x_ref[0]
    keys = jnp.dot(xb, wkb_ref[...], preferred_element_type=jnp.float32) + bkb_ref[0]
    qs = jnp.dot(xb, wqb_ref[...], preferred_element_type=jnp.float32) + bqb_ref[0]
    keys_prev = pltpu.roll(keys, 1, 0)
    z = jnp.sum(keys_prev * qs, axis=1, keepdims=True)
    rows = jax.lax.broadcasted_iota(jnp.int32, (T, T), 0)
    cols = jax.lax.broadcasted_iota(jnp.int32, (T, T), 1)
    row_idx = jax.lax.broadcasted_iota(jnp.int32, (T, 1), 0)
    bmask = jnp.where((z > 0.0) & (row_idx > 0), 1.0, 0.0)
    starts = 1.0 - bmask
    lower = (rows <= cols).astype(jnp.float32)
    seg = jnp.sum(starts * lower, axis=0, keepdims=True) - 1.0
    seg_i = seg.astype(jnp.int32)
    s_mat = (rows == seg_i).astype(jnp.float32)
    counts = jnp.sum(s_mat, axis=1, keepdims=True)
    sums = jnp.dot(s_mat, xb, preferred_element_type=jnp.float32)
    means = sums / jnp.maximum(counts, 1.0)
    h = _ln(jnp.dot(means, w1_ref[...], preferred_element_type=jnp.float32)
            + b1_ref[0], g_ref[0], bn_ref[0])
    h = jnp.maximum(h, 0.0)
    out = jnp.dot(h, w2_ref[...], preferred_element_type=jnp.float32) + b2_ref[0]
    valid = (counts > 0.0).astype(jnp.float32)
    ch_ref[0] = out * valid
    pad_ref[0] = 1.0 - valid.reshape(1, T)
    s_ref[0] = s_mat


def _bind_agg(x, bp, ap):
    row = lambda a: a.reshape(1, -1)
    const = lambda shape: pl.BlockSpec(shape, lambda b: (0,) * len(shape))
    ch, pad, s_mat = pl.pallas_call(
        _bind_agg_kernel,
        grid=(B,),
        in_specs=[
            pl.BlockSpec((1, T, D), lambda b: (b, 0, 0)),
            const((D, D // 2)), const((1, D // 2)),
            const((D, D // 2)), const((1, D // 2)),
            const((D, D)), const((1, D)),
            const((1, D)), const((1, D)),
            const((D, D)), const((1, D)),
        ],
        out_specs=[
            pl.BlockSpec((1, T, D), lambda b: (b, 0, 0)),
            pl.BlockSpec((1, 1, T), lambda b: (b, 0, 0)),
            pl.BlockSpec((1, T, T), lambda b: (b, 0, 0)),
        ],
        out_shape=[
            jax.ShapeDtypeStruct((B, T, D), jnp.float32),
            jax.ShapeDtypeStruct((B, 1, T), jnp.float32),
            jax.ShapeDtypeStruct((B, T, T), jnp.float32),
        ],
    )(x, bp['Wk'], row(bp['bk']), bp['Wq'], row(bp['bq']),
      ap['W1'], row(ap['b1']), row(ap['g']), row(ap['bn']),
      ap['W2'], row(ap['b2']))
    return ch, pad, s_mat


def _down_kernel(h_ref, l_ref, s_ref, w_ref, b_ref, out_ref):
    hb = h_ref[0]
    lb = l_ref[0]
    sb = s_ref[0]
    infl = jnp.dot(hb, w_ref[...], preferred_element_type=jnp.float32) + b_ref[0]
    infl_exp = jax.lax.dot_general(sb, infl, (((0,), (0,)), ((), ())),
                                   preferred_element_type=jnp.float32)
    gate = jax.nn.sigmoid(infl_exp)
    out_ref[0] = lb * gate + lb


def _down(higher, lower, s_mat, p):
    row = lambda a: a.reshape(1, -1)
    const = lambda shape: pl.BlockSpec(shape, lambda b: (0,) * len(shape))
    return pl.pallas_call(
        _down_kernel,
        grid=(B,),
        in_specs=[
            pl.BlockSpec((1, T, D), lambda b: (b, 0, 0)),
            pl.BlockSpec((1, T, D), lambda b: (b, 0, 0)),
            pl.BlockSpec((1, T, T), lambda b: (b, 0, 0)),
            const((D, D)), const((1, D)),
        ],
        out_specs=pl.BlockSpec((1, T, D), lambda b: (b, 0, 0)),
        out_shape=jax.ShapeDtypeStruct((B, T, D), jnp.float32),
    )(higher, lower, s_mat, p['W'], row(p['b']))


def kernel(x, params):
    scales = params['scales']
    binds = params['binds']
    aggs = params['aggs']
    downs = params['downs']
    zero_mask = jnp.zeros((B, 1, T), dtype=jnp.float32)
    rep0 = _scale_block(x, scales[0], zero_mask)
    ch0, pad0, s0 = _bind_agg(rep0, binds[0], aggs[0])
    rep1 = _scale_block(ch0, scales[1], pad0)
    ch1, pad1, s1 = _bind_agg(rep1, binds[1], aggs[1])
    rep2 = _scale_block(ch1, scales[2], pad1)
    rep1 = _down(rep2, rep1, s1, downs[1])
    rep0 = _down(rep1, rep0, s0, downs[0])
    return rep0
